# baseline (speedup 1.0000x reference)
import functools

import jax
import jax.numpy as jnp
from jax import lax
from jax.experimental import pallas as pl
from jax.experimental.pallas import tpu as pltpu
from jax.experimental.pallas import tpu_sc as plsc

VOCAB = 100000
EMBED_DIM = 128
BATCH = 4096
HIST_LEN = 50

B_TOTAL = BATCH * HIST_LEN
NUM_CORES = 2
NUM_SUBCORES = 16
NW = NUM_CORES * NUM_SUBCORES
BATCH_PER_W = BATCH // NW
B_PER_W = B_TOTAL // NW

ENT_PER_CHUNK = 8
CHUNK = ENT_PER_CHUNK * HIST_LEN
N_CHUNKS = BATCH_PER_W // ENT_PER_CHUNK
SUB = 80
N_SUB = CHUNK // SUB
NBUF = 2
G = 1


def _gather_body(idx_hbm, table_hbm, out_hbm, idx_v, rows_v, gsems, wsems):
    wid = lax.axis_index("s") * NUM_CORES + lax.axis_index("c")
    base = wid * B_PER_W
    ent_base = wid * BATCH_PER_W
    pltpu.sync_copy(idx_hbm.at[pl.ds(base, B_PER_W)], idx_v)

    def g_start(c, b):
        for k in range(N_SUB):
            pltpu.make_async_copy(
                table_hbm.at[idx_v.at[pl.ds(c * CHUNK + k * SUB, SUB)]],
                rows_v.at[b].at[pl.ds(k * SUB, SUB)],
                gsems.at[b],
            ).start()

    def g_wait(c, b):
        for k in range(N_SUB):
            pltpu.make_async_copy(
                table_hbm.at[idx_v.at[pl.ds(c * CHUNK + k * SUB, SUB)]],
                rows_v.at[b].at[pl.ds(k * SUB, SUB)],
                gsems.at[b],
            ).wait()

    def w_copy(c, b):
        return pltpu.make_async_copy(
            rows_v.at[b].reshape(ENT_PER_CHUNK, HIST_LEN, EMBED_DIM),
            out_hbm.at[pl.ds(ent_base + c * ENT_PER_CHUNK, ENT_PER_CHUNK)],
            wsems.at[b],
        )

    for b in range(G):
        g_start(b, b)

    @pl.loop(0, N_CHUNKS, step=NBUF)
    def _outer(g):
        for b in range(NBUF):
            c = g + b
            g_wait(c, b)
            w_copy(c, b).start()
            nb = (b + G) % NBUF

            @pl.when(c + G < N_CHUNKS)
            def _start_next():
                @pl.when(c >= NBUF - G)
                def _free_slot():
                    w_copy(c - (NBUF - G), nb).wait()

                g_start(c + G, nb)

    for b in range(NBUF):
        w_copy(N_CHUNKS - NBUF + b, b).wait()


_kernel_call = functools.partial(
    pl.kernel,
    out_type=jax.ShapeDtypeStruct((BATCH, HIST_LEN, EMBED_DIM), jnp.float32),
    mesh=plsc.VectorSubcoreMesh(
        core_axis_name="c", subcore_axis_name="s",
        num_cores=NUM_CORES, num_subcores=NUM_SUBCORES,
    ),
    scratch_types=[
        pltpu.VMEM((B_PER_W,), jnp.int32),
        pltpu.VMEM((NBUF, CHUNK, EMBED_DIM), jnp.float32),
        pltpu.SemaphoreType.DMA((NBUF,)),
        pltpu.SemaphoreType.DMA((NBUF,)),
    ],
    compiler_params=pltpu.CompilerParams(use_tc_tiling_on_sc=True),
)(_gather_body)


@jax.jit
def kernel(hidden_states, emb_table):
    flat_idx = hidden_states.reshape(B_TOTAL)
    return _kernel_call(flat_idx, emb_table)


# speedup vs baseline: 1.7984x; 1.7984x over previous
import functools

import jax
import jax.numpy as jnp
from jax import lax
from jax.experimental import pallas as pl
from jax.experimental.pallas import tpu as pltpu
from jax.experimental.pallas import tpu_sc as plsc

VOCAB = 100000
EMBED_DIM = 128
BATCH = 4096
HIST_LEN = 50

B_TOTAL = BATCH * HIST_LEN
NUM_CORES = 2
NUM_SUBCORES = 16
NW = NUM_CORES * NUM_SUBCORES
B_PER_W = B_TOTAL // NW
CHUNK = 128
N_CHUNKS = B_PER_W // CHUNK
NBUF = 5
G = 2


def _gather_body(idx_hbm, table_hbm, out_hbm, idx_v, rows_v, gsems, wsems):
    wid = lax.axis_index("s") * NUM_CORES + lax.axis_index("c")
    base = wid * B_PER_W
    pltpu.sync_copy(idx_hbm.at[pl.ds(base, B_PER_W)], idx_v)

    def g_copy(c, b):
        return pltpu.make_async_copy(
            table_hbm.at[idx_v.at[pl.ds(c * CHUNK, CHUNK)]],
            rows_v.at[b],
            gsems.at[b],
        )

    def w_copy(c, b):
        return pltpu.make_async_copy(
            rows_v.at[b],
            out_hbm.at[pl.ds(base + c * CHUNK, CHUNK)],
            wsems.at[b],
        )

    for b in range(G):
        g_copy(b, b).start()

    @pl.loop(0, N_CHUNKS, step=NBUF)
    def _outer(g):
        for b in range(NBUF):
            c = g + b
            g_copy(c, b).wait()
            w_copy(c, b).start()
            nb = (b + G) % NBUF

            @pl.when(c + G < N_CHUNKS)
            def _start_next():
                @pl.when(c >= NBUF - G)
                def _free_slot():
                    w_copy(c - (NBUF - G), nb).wait()

                g_copy(c + G, nb).start()

    for b in range(NBUF):
        w_copy(N_CHUNKS - NBUF + b, b).wait()


_kernel_call = functools.partial(
    pl.kernel,
    out_type=jax.ShapeDtypeStruct((B_TOTAL, EMBED_DIM), jnp.float32),
    mesh=plsc.VectorSubcoreMesh(
        core_axis_name="c", subcore_axis_name="s",
        num_cores=NUM_CORES, num_subcores=NUM_SUBCORES,
    ),
    scratch_types=[
        pltpu.VMEM((B_PER_W,), jnp.int32),
        pltpu.VMEM((NBUF, CHUNK, EMBED_DIM), jnp.float32),
        pltpu.SemaphoreType.DMA((NBUF,)),
        pltpu.SemaphoreType.DMA((NBUF,)),
    ],
)(_gather_body)


@jax.jit
def kernel(hidden_states, emb_table):
    flat_idx = hidden_states.T.reshape(B_TOTAL)
    out = _kernel_call(flat_idx, emb_table)
    return out.reshape(HIST_LEN, BATCH, EMBED_DIM).transpose(1, 0, 2)
